# Initial kernel scaffold; baseline (speedup 1.0000x reference)
import jax
import jax.numpy as jnp
from jax.experimental import pallas as pl


def kernel(x, edge_index, batch, trackster_features, params):
    raise NotImplementedError("write your pallas kernel here")


# speedup vs baseline: 31.0779x; 31.0779x over previous
import functools

import jax
import jax.numpy as jnp
from jax import lax
from jax.experimental import pallas as pl
from jax.experimental.pallas import tpu as pltpu
from jax.experimental.pallas import tpu_sc as plsc

N = 50000
E = 800000
B = 64
F_IN = 128
H = 64
HEADS = 4
DH = 16
NC = 8

NCORE = 2
NSUB = 16
HALF = N // NCORE
EPW = E // NSUB

WACC = 72
ACC_ROWS = 25088
SLAB = ACC_ROWS // NSUB
DUMMY = 25080
CAP = 51200
CH_P = 2000
CH_E = 80

BLK = 1000
GRID = N // BLK

def _edge_body(hp_hbm, ad_hbm, src_hbm, dst_hbm, zeros_hbm, out_hbm,
               acc, sidx, dbuf, didx, hpbuf, adbuf, outbuf,
               sem1, sem2):
    c = lax.axis_index("c")
    s = lax.axis_index("s")
    lo = c * HALF

    pltpu.sync_copy(zeros_hbm, acc.at[pl.ds(s * SLAB, SLAB)])
    plsc.subcore_barrier()

    @pl.loop(0, E // (NSUB * CH_E))
    def _trip(t):
        off = (t * NSUB + s) * CH_E
        pltpu.sync_copy(src_hbm.at[pl.ds(off, CH_E)], sidx)
        pltpu.sync_copy(dst_hbm.at[pl.ds(off, CH_E)], dbuf)
        for g in range(CH_E // 16):
            dv = dbuf[pl.ds(g * 16, 16)]
            inh = (dv >= lo) & (dv < lo + HALF)
            didx[pl.ds(g * 16, 16)] = jnp.where(inh, dv - lo, DUMMY)
        pltpu.async_copy(hp_hbm.at[sidx], hpbuf, sem1).wait()
        pltpu.async_copy(ad_hbm.at[dbuf], adbuf, sem2).wait()
        iot = lax.iota(jnp.int32, 16)
        for g in range(CH_E // 16):
            rows = iot + g * 16
            for j in range(HEADS):
                asv = plsc.load_gather(
                    hpbuf, [rows, jnp.full((16,), H + j, jnp.int32)])
                adv = plsc.load_gather(
                    adbuf, [rows, jnp.full((16,), j, jnp.int32)])
                e = asv + adv
                e = jnp.where(e >= 0.0, e, 0.2 * e)
                ex = jnp.exp(e)
                plsc.store_scatter(
                    outbuf, [rows, jnp.full((16,), H + j, jnp.int32)], ex)

        @pl.loop(0, CH_E)
        def _scale(ei):
            tailv = outbuf[ei, pl.ds(WACC - 16, 16)]
            for j in range(HEADS):
                exs = tailv[H + j - (WACC - 16)]
                outbuf[ei, pl.ds(DH * j, DH)] = (
                    hpbuf[ei, pl.ds(DH * j, DH)] * exs)

        pltpu.sync_copy(outbuf, acc.at[didx], add=True)

    plsc.subcore_barrier()

    @pl.when(s < NSUB - 1)
    def _():
        pltpu.sync_copy(acc.at[pl.ds(s * SLAB, SLAB)],
                        out_hbm.at[pl.ds(c * HALF + s * SLAB, SLAB)])

    @pl.when(s == NSUB - 1)
    def _():
        pltpu.sync_copy(acc.at[pl.ds((NSUB - 1) * SLAB, HALF - (NSUB - 1) * SLAB)],
                        out_hbm.at[pl.ds(c * HALF + (NSUB - 1) * SLAB,
                                         HALF - (NSUB - 1) * SLAB)])


@functools.cache
def _get_edge():
    mesh = plsc.VectorSubcoreMesh(core_axis_name="c", subcore_axis_name="s",
                                  num_cores=NCORE, num_subcores=NSUB)
    return pl.kernel(
        _edge_body,
        out_type=jax.ShapeDtypeStruct((N, WACC), jnp.float32),
        mesh=mesh,
        compiler_params=pltpu.CompilerParams(needs_layout_passes=False,
                                             use_tc_tiling_on_sc=False),
        scratch_types=[
            pltpu.VMEM_SHARED((ACC_ROWS, WACC), jnp.float32),
            pltpu.VMEM((CH_E,), jnp.int32),
            pltpu.VMEM((CH_E,), jnp.int32),
            pltpu.VMEM((CH_E,), jnp.int32),
            pltpu.VMEM((CH_E, H + DH), jnp.float32),
            pltpu.VMEM((CH_E, DH), jnp.float32),
            pltpu.VMEM((CH_E, WACC), jnp.float32),
            pltpu.SemaphoreType.DMA,
            pltpu.SemaphoreType.DMA,
        ],
    )


def _heads_expand(v4):
    return jnp.concatenate(
        [jnp.broadcast_to(v4[:, j:j + 1], (v4.shape[0], DH)) for j in
         range(HEADS)], axis=1)


def _project(h, w_ref, as_ref, ad_ref, hpe_ref, atd_ref):
    hp = jnp.dot(h, w_ref[...], preferred_element_type=jnp.float32)
    a_s = jnp.dot(hp, as_ref[...], preferred_element_type=jnp.float32)
    a_d = jnp.dot(hp, ad_ref[...], preferred_element_type=jnp.float32)
    hpe_ref[...] = jnp.concatenate(
        [hp, a_s, jnp.zeros((BLK, DH - HEADS), jnp.float32)], axis=1)
    atd_ref[...] = jnp.concatenate(
        [a_d, jnp.zeros((BLK, DH - HEADS), jnp.float32)], axis=1)


def _prologue_body(x_ref, ew_ref, eb_ref, w_ref, as_ref, ad_ref,
                   hpe_ref, atd_ref):
    h0 = jnp.maximum(
        jnp.dot(x_ref[...], ew_ref[...], preferred_element_type=jnp.float32)
        + eb_ref[...], 0.0)
    _project(h0, w_ref, as_ref, ad_ref, hpe_ref, atd_ref)


def _mid_body(un_ref, b_ref, w_ref, as_ref, ad_ref, hpe_ref, atd_ref):
    un = un_ref[...]
    rec = 1.0 / (un[:, H:H + HEADS] + 1e-16)
    h = jnp.maximum(un[:, :H] * _heads_expand(rec) + b_ref[...], 0.0)
    _project(h, w_ref, as_ref, ad_ref, hpe_ref, atd_ref)


def _wspec(shape):
    return pl.BlockSpec(shape, lambda i: (0,) * len(shape))


_prologue = pl.pallas_call(
    _prologue_body,
    grid=(GRID,),
    in_specs=[
        pl.BlockSpec((BLK, F_IN), lambda i: (i, 0)),
        _wspec((F_IN, H)), _wspec((1, H)), _wspec((H, H)),
        _wspec((H, HEADS)), _wspec((H, HEADS)),
    ],
    out_specs=[
        pl.BlockSpec((BLK, H + DH), lambda i: (i, 0)),
        pl.BlockSpec((BLK, DH), lambda i: (i, 0)),
    ],
    out_shape=[
        jax.ShapeDtypeStruct((N, H + DH), jnp.float32),
        jax.ShapeDtypeStruct((N, DH), jnp.float32),
    ],
)

_mid = pl.pallas_call(
    _mid_body,
    grid=(GRID,),
    in_specs=[
        pl.BlockSpec((BLK, WACC), lambda i: (i, 0)),
        _wspec((1, H)), _wspec((H, H)),
        _wspec((H, HEADS)), _wspec((H, HEADS)),
    ],
    out_specs=[
        pl.BlockSpec((BLK, H + DH), lambda i: (i, 0)),
        pl.BlockSpec((BLK, DH), lambda i: (i, 0)),
    ],
    out_shape=[
        jax.ShapeDtypeStruct((N, H + DH), jnp.float32),
        jax.ShapeDtypeStruct((N, DH), jnp.float32),
    ],
)


def _final_body(un_ref, b_ref, bf_ref, sums_ref, maxs_ref, cnts_ref):
    i = pl.program_id(0)

    @pl.when(i == 0)
    def _():
        sums_ref[...] = jnp.zeros((B, H), jnp.float32)
        maxs_ref[...] = jnp.full((B, H), -jnp.inf, jnp.float32)
        cnts_ref[...] = jnp.zeros((B, H), jnp.float32)

    un = un_ref[...]
    rec = 1.0 / (un[:, H:H + HEADS] + 1e-16)
    h = jnp.maximum(un[:, :H] * _heads_expand(rec) + b_ref[...], 0.0)
    bf = bf_ref[...]
    blo = bf_ref[0, 0]
    bhi = bf_ref[BLK - 1, 0]
    for b in range(B):
        @pl.when((blo <= float(b)) & (float(b) <= bhi))
        def _():
            m = bf == float(b)
            mb = jnp.max(jnp.where(m, h, -jnp.inf), axis=0, keepdims=True)
            maxs_ref[b:b + 1, :] = jnp.maximum(maxs_ref[b:b + 1, :], mb)
            sums_ref[b:b + 1, :] += jnp.sum(
                jnp.where(m, h, 0.0), axis=0, keepdims=True)
            cnts_ref[b:b + 1, :] += jnp.full(
                (1, H), jnp.sum(m.astype(jnp.float32)))


_final = pl.pallas_call(
    _final_body,
    grid=(GRID,),
    in_specs=[
        pl.BlockSpec((BLK, WACC), lambda i: (i, 0)),
        _wspec((1, H)),
        pl.BlockSpec((BLK, 1), lambda i: (i, 0)),
    ],
    out_specs=[_wspec((B, H)), _wspec((B, H)), _wspec((B, H))],
    out_shape=[
        jax.ShapeDtypeStruct((B, H), jnp.float32),
        jax.ShapeDtypeStruct((B, H), jnp.float32),
        jax.ShapeDtypeStruct((B, H), jnp.float32),
    ],
)


def _ln(x, g, b):
    m = jnp.mean(x, axis=-1, keepdims=True)
    v = jnp.mean((x - m) ** 2, axis=-1, keepdims=True)
    return (x - m) / jnp.sqrt(v + 1e-5) * g + b


def _head_body(sums_ref, maxs_ref, cnts_ref, ts_ref,
               tw1_ref, tb1_ref, tg_ref, tbb_ref, tw2_ref, tb2_ref,
               cg_ref, cb_ref, cw1_ref, cb1_ref, cw2_ref, cb2_ref,
               out_ref):
    mean = sums_ref[...] / jnp.clip(cnts_ref[...], 1.0, None)
    t = jnp.dot(ts_ref[...], tw1_ref[...],
                preferred_element_type=jnp.float32) + tb1_ref[...]
    t = _ln(t, tg_ref[...], tbb_ref[...])
    t = jnp.maximum(t, 0.0)
    t = jnp.dot(t, tw2_ref[...],
                preferred_element_type=jnp.float32) + tb2_ref[...]
    pooled = jnp.concatenate([mean, maxs_ref[...], t], axis=1)
    z = _ln(pooled, cg_ref[...], cb_ref[...])
    z = jnp.maximum(
        jnp.dot(z, cw1_ref[...], preferred_element_type=jnp.float32)
        + cb1_ref[...], 0.0)
    out_ref[...] = jnp.dot(
        z, cw2_ref[...], preferred_element_type=jnp.float32) + cb2_ref[...]


_POOL = 2 * H + H // 2

_head = pl.pallas_call(
    _head_body,
    out_shape=jax.ShapeDtypeStruct((B, NC), jnp.float32),
)


def _att_mat(att):
    return (att[:, :, None] * jnp.eye(HEADS, dtype=jnp.float32)[:, None, :]
            ).reshape(H, HEADS)


def kernel(x, edge_index, batch, trackster_features, params):
    p = params
    src, dst = edge_index[0], edge_index[1]
    zrows = jnp.zeros((SLAB, WACC), jnp.float32)
    edge_call = _get_edge()

    convs = p['convs']
    as_m = [_att_mat(c['att_src']) for c in convs]
    ad_m = [_att_mat(c['att_dst']) for c in convs]

    hpe, atd = _prologue(x, p['enc_W'], p['enc_b'].reshape(1, H),
                         convs[0]['W'], as_m[0], ad_m[0])
    un = None
    for L in range(3):
        un = edge_call(hpe, atd, src, dst, zrows)
        if L < 2:
            hpe, atd = _mid(un, convs[L]['b'].reshape(1, H),
                            convs[L + 1]['W'], as_m[L + 1], ad_m[L + 1])

    sums, maxs, cnts = _final(un, convs[2]['b'].reshape(1, H),
                              batch.astype(jnp.float32).reshape(N, 1))

    return _head(
        sums, maxs, cnts, trackster_features,
        p['ts_W1'], p['ts_b1'].reshape(1, H // 2),
        p['ts_ln_g'].reshape(1, H // 2), p['ts_ln_b'].reshape(1, H // 2),
        p['ts_W2'], p['ts_b2'].reshape(1, H // 2),
        p['cl_ln_g'].reshape(1, _POOL), p['cl_ln_b'].reshape(1, _POOL),
        p['cl_W1'], p['cl_b1'].reshape(1, H),
        p['cl_W2'], p['cl_b2'].reshape(1, NC),
    )


# speedup vs baseline: 91.2140x; 2.9350x over previous
import functools

import jax
import jax.numpy as jnp
from jax import lax
from jax.experimental import pallas as pl
from jax.experimental.pallas import tpu as pltpu
from jax.experimental.pallas import tpu_sc as plsc

N = 50000
E = 800000
B = 64
F_IN = 128
H = 64
HEADS = 4
DH = 16
NC = 8

NCORE = 2
NSUB = 16
HALF = N // NCORE
EPW = E // NSUB

WACC = 72
ACC_ROWS = 25088
SLAB = ACC_ROWS // NSUB
DUMMY = 25080
CH_E = 80
ST = 400
SUB = ST // CH_E
ROUNDS = E // (NSUB * ST)

BLK = 1000
GRID = N // BLK

def _edge_body(hp_hbm, ad_hbm, src_hbm, dst_hbm, zeros_hbm, out_hbm,
               acc, sbuf, dbuf, hp0, hp1, ad0, ad1, di0, di1,
               sg0, sg1, ss0, ss1):
    c = lax.axis_index("c")
    s = lax.axis_index("s")
    lo = c * HALF

    pltpu.sync_copy(zeros_hbm, acc.at[pl.ds(s * SLAB, SLAB)])
    plsc.subcore_barrier()

    hpb = (hp0, hp1)
    adb = (ad0, ad1)
    dib = (di0, di1)
    sgs = (sg0, sg1)
    sss = (ss0, ss1)
    iot = lax.iota(jnp.int32, 16)

    def fire_gathers(k, p):
        g = pltpu.async_copy(hp_hbm.at[sbuf.at[pl.ds(k * CH_E, CH_E)]],
                             hpb[p], sgs[p])
        a = pltpu.async_copy(ad_hbm.at[dbuf.at[pl.ds(k * CH_E, CH_E)]],
                             adb[p], sgs[p])
        return g, a

    @pl.loop(0, ROUNDS)
    def _round(r):
        off = (r * NSUB + s) * ST
        pltpu.sync_copy(src_hbm.at[pl.ds(off, ST)], sbuf)
        pltpu.sync_copy(dst_hbm.at[pl.ds(off, ST)], dbuf)

        gat = fire_gathers(0, 0)
        scat = [None, None]
        for k in range(SUB):
            p = k % 2
            if k + 1 < SUB:
                if scat[1 - p] is not None:
                    scat[1 - p].wait()
                    scat[1 - p] = None
                nxt = fire_gathers(k + 1, 1 - p)
            gat[0].wait()
            gat[1].wait()
            buf = hpb[p]
            for g in range(CH_E // 16):
                dv = dbuf[pl.ds(k * CH_E + g * 16, 16)]
                inh = (dv >= lo) & (dv < lo + HALF)
                dib[p][pl.ds(g * 16, 16)] = jnp.where(inh, dv - lo, DUMMY)
            for g in range(CH_E // 16):
                rows = iot + g * 16
                for j in range(HEADS):
                    asv = plsc.load_gather(
                        buf, [rows, jnp.full((16,), H + j, jnp.int32)])
                    adv = plsc.load_gather(
                        adb[p], [rows, jnp.full((16,), j, jnp.int32)])
                    e = asv + adv
                    e = jnp.where(e >= 0.0, e, 0.2 * e)
                    ex = jnp.exp(e)
                    plsc.store_scatter(
                        buf, [rows, jnp.full((16,), H + j, jnp.int32)], ex)

            @pl.loop(0, CH_E, unroll=4)
            def _scale(ei):
                tailv = buf[ei, pl.ds(WACC - 16, 16)]
                for j in range(HEADS):
                    exs = tailv[H + j - (WACC - 16)]
                    buf[ei, pl.ds(DH * j, DH)] = (
                        buf[ei, pl.ds(DH * j, DH)] * exs)

            if scat[p] is not None:
                scat[p].wait()
            scat[p] = pltpu.async_copy(buf, acc.at[dib[p]], sss[p], add=True)
            if k + 1 < SUB:
                gat = nxt
        for d in scat:
            if d is not None:
                d.wait()

    plsc.subcore_barrier()

    @pl.when(s < NSUB - 1)
    def _():
        pltpu.sync_copy(acc.at[pl.ds(s * SLAB, SLAB)],
                        out_hbm.at[pl.ds(c * HALF + s * SLAB, SLAB)])

    @pl.when(s == NSUB - 1)
    def _():
        pltpu.sync_copy(acc.at[pl.ds((NSUB - 1) * SLAB, HALF - (NSUB - 1) * SLAB)],
                        out_hbm.at[pl.ds(c * HALF + (NSUB - 1) * SLAB,
                                         HALF - (NSUB - 1) * SLAB)])


@functools.cache
def _get_edge():
    mesh = plsc.VectorSubcoreMesh(core_axis_name="c", subcore_axis_name="s",
                                  num_cores=NCORE, num_subcores=NSUB)
    return pl.kernel(
        _edge_body,
        out_type=jax.ShapeDtypeStruct((N, WACC), jnp.float32),
        mesh=mesh,
        compiler_params=pltpu.CompilerParams(needs_layout_passes=False,
                                             use_tc_tiling_on_sc=False),
        scratch_types=[
            pltpu.VMEM_SHARED((ACC_ROWS, WACC), jnp.float32),
            pltpu.VMEM((ST,), jnp.int32),
            pltpu.VMEM((ST,), jnp.int32),
            pltpu.VMEM((CH_E, WACC), jnp.float32),
            pltpu.VMEM((CH_E, WACC), jnp.float32),
            pltpu.VMEM((CH_E, DH), jnp.float32),
            pltpu.VMEM((CH_E, DH), jnp.float32),
            pltpu.VMEM((CH_E,), jnp.int32),
            pltpu.VMEM((CH_E,), jnp.int32),
            pltpu.SemaphoreType.DMA,
            pltpu.SemaphoreType.DMA,
            pltpu.SemaphoreType.DMA,
            pltpu.SemaphoreType.DMA,
        ],
    )


def _heads_expand(v4):
    return jnp.concatenate(
        [jnp.broadcast_to(v4[:, j:j + 1], (v4.shape[0], DH)) for j in
         range(HEADS)], axis=1)


def _project(h, w_ref, as_ref, ad_ref, hpe_ref, atd_ref):
    hp = jnp.dot(h, w_ref[...], preferred_element_type=jnp.float32)
    a_s = jnp.dot(hp, as_ref[...], preferred_element_type=jnp.float32)
    a_d = jnp.dot(hp, ad_ref[...], preferred_element_type=jnp.float32)
    hpe_ref[...] = jnp.concatenate(
        [hp, a_s, jnp.zeros((BLK, WACC - H - HEADS), jnp.float32)], axis=1)
    atd_ref[...] = jnp.concatenate(
        [a_d, jnp.zeros((BLK, DH - HEADS), jnp.float32)], axis=1)


def _prologue_body(x_ref, ew_ref, eb_ref, w_ref, as_ref, ad_ref,
                   hpe_ref, atd_ref):
    h0 = jnp.maximum(
        jnp.dot(x_ref[...], ew_ref[...], preferred_element_type=jnp.float32)
        + eb_ref[...], 0.0)
    _project(h0, w_ref, as_ref, ad_ref, hpe_ref, atd_ref)


def _mid_body(un_ref, b_ref, w_ref, as_ref, ad_ref, hpe_ref, atd_ref):
    un = un_ref[...]
    rec = 1.0 / (un[:, H:H + HEADS] + 1e-16)
    h = jnp.maximum(un[:, :H] * _heads_expand(rec) + b_ref[...], 0.0)
    _project(h, w_ref, as_ref, ad_ref, hpe_ref, atd_ref)


def _wspec(shape):
    return pl.BlockSpec(shape, lambda i: (0,) * len(shape))


_prologue = pl.pallas_call(
    _prologue_body,
    grid=(GRID,),
    in_specs=[
        pl.BlockSpec((BLK, F_IN), lambda i: (i, 0)),
        _wspec((F_IN, H)), _wspec((1, H)), _wspec((H, H)),
        _wspec((H, HEADS)), _wspec((H, HEADS)),
    ],
    out_specs=[
        pl.BlockSpec((BLK, WACC), lambda i: (i, 0)),
        pl.BlockSpec((BLK, DH), lambda i: (i, 0)),
    ],
    out_shape=[
        jax.ShapeDtypeStruct((N, WACC), jnp.float32),
        jax.ShapeDtypeStruct((N, DH), jnp.float32),
    ],
)

_mid = pl.pallas_call(
    _mid_body,
    grid=(GRID,),
    in_specs=[
        pl.BlockSpec((BLK, WACC), lambda i: (i, 0)),
        _wspec((1, H)), _wspec((H, H)),
        _wspec((H, HEADS)), _wspec((H, HEADS)),
    ],
    out_specs=[
        pl.BlockSpec((BLK, WACC), lambda i: (i, 0)),
        pl.BlockSpec((BLK, DH), lambda i: (i, 0)),
    ],
    out_shape=[
        jax.ShapeDtypeStruct((N, WACC), jnp.float32),
        jax.ShapeDtypeStruct((N, DH), jnp.float32),
    ],
)


def _final_body(un_ref, b_ref, bf_ref, sums_ref, maxs_ref, cnts_ref):
    i = pl.program_id(0)

    @pl.when(i == 0)
    def _():
        sums_ref[...] = jnp.zeros((B, H), jnp.float32)
        maxs_ref[...] = jnp.full((B, H), -jnp.inf, jnp.float32)
        cnts_ref[...] = jnp.zeros((B, H), jnp.float32)

    un = un_ref[...]
    rec = 1.0 / (un[:, H:H + HEADS] + 1e-16)
    h = jnp.maximum(un[:, :H] * _heads_expand(rec) + b_ref[...], 0.0)
    bf = bf_ref[...]
    blo = bf_ref[0, 0]
    bhi = bf_ref[BLK - 1, 0]
    for b in range(B):
        @pl.when((blo <= float(b)) & (float(b) <= bhi))
        def _():
            m = bf == float(b)
            mb = jnp.max(jnp.where(m, h, -jnp.inf), axis=0, keepdims=True)
            maxs_ref[b:b + 1, :] = jnp.maximum(maxs_ref[b:b + 1, :], mb)
            sums_ref[b:b + 1, :] += jnp.sum(
                jnp.where(m, h, 0.0), axis=0, keepdims=True)
            cnts_ref[b:b + 1, :] += jnp.full(
                (1, H), jnp.sum(m.astype(jnp.float32)))


_final = pl.pallas_call(
    _final_body,
    grid=(GRID,),
    in_specs=[
        pl.BlockSpec((BLK, WACC), lambda i: (i, 0)),
        _wspec((1, H)),
        pl.BlockSpec((BLK, 1), lambda i: (i, 0)),
    ],
    out_specs=[_wspec((B, H)), _wspec((B, H)), _wspec((B, H))],
    out_shape=[
        jax.ShapeDtypeStruct((B, H), jnp.float32),
        jax.ShapeDtypeStruct((B, H), jnp.float32),
        jax.ShapeDtypeStruct((B, H), jnp.float32),
    ],
)


def _ln(x, g, b):
    m = jnp.mean(x, axis=-1, keepdims=True)
    v = jnp.mean((x - m) ** 2, axis=-1, keepdims=True)
    return (x - m) / jnp.sqrt(v + 1e-5) * g + b


def _head_body(sums_ref, maxs_ref, cnts_ref, ts_ref,
               tw1_ref, tb1_ref, tg_ref, tbb_ref, tw2_ref, tb2_ref,
               cg_ref, cb_ref, cw1_ref, cb1_ref, cw2_ref, cb2_ref,
               out_ref):
    mean = sums_ref[...] / jnp.clip(cnts_ref[...], 1.0, None)
    t = jnp.dot(ts_ref[...], tw1_ref[...],
                preferred_element_type=jnp.float32) + tb1_ref[...]
    t = _ln(t, tg_ref[...], tbb_ref[...])
    t = jnp.maximum(t, 0.0)
    t = jnp.dot(t, tw2_ref[...],
                preferred_element_type=jnp.float32) + tb2_ref[...]
    pooled = jnp.concatenate([mean, maxs_ref[...], t], axis=1)
    z = _ln(pooled, cg_ref[...], cb_ref[...])
    z = jnp.maximum(
        jnp.dot(z, cw1_ref[...], preferred_element_type=jnp.float32)
        + cb1_ref[...], 0.0)
    out_ref[...] = jnp.dot(
        z, cw2_ref[...], preferred_element_type=jnp.float32) + cb2_ref[...]


_POOL = 2 * H + H // 2

_head = pl.pallas_call(
    _head_body,
    out_shape=jax.ShapeDtypeStruct((B, NC), jnp.float32),
)


def _att_mat(att):
    return (att[:, :, None] * jnp.eye(HEADS, dtype=jnp.float32)[:, None, :]
            ).reshape(H, HEADS)


def kernel(x, edge_index, batch, trackster_features, params):
    p = params
    src, dst = edge_index[0], edge_index[1]
    zrows = jnp.zeros((SLAB, WACC), jnp.float32)
    edge_call = _get_edge()

    convs = p['convs']
    as_m = [_att_mat(c['att_src']) for c in convs]
    ad_m = [_att_mat(c['att_dst']) for c in convs]

    hpe, atd = _prologue(x, p['enc_W'], p['enc_b'].reshape(1, H),
                         convs[0]['W'], as_m[0], ad_m[0])
    un = None
    for L in range(3):
        un = edge_call(hpe, atd, src, dst, zrows)
        if L < 2:
            hpe, atd = _mid(un, convs[L]['b'].reshape(1, H),
                            convs[L + 1]['W'], as_m[L + 1], ad_m[L + 1])

    sums, maxs, cnts = _final(un, convs[2]['b'].reshape(1, H),
                              batch.astype(jnp.float32).reshape(N, 1))

    return _head(
        sums, maxs, cnts, trackster_features,
        p['ts_W1'], p['ts_b1'].reshape(1, H // 2),
        p['ts_ln_g'].reshape(1, H // 2), p['ts_ln_b'].reshape(1, H // 2),
        p['ts_W2'], p['ts_b2'].reshape(1, H // 2),
        p['cl_ln_g'].reshape(1, _POOL), p['cl_ln_b'].reshape(1, _POOL),
        p['cl_W1'], p['cl_b1'].reshape(1, H),
        p['cl_W2'], p['cl_b2'].reshape(1, NC),
    )


# speedup vs baseline: 130.7785x; 1.4338x over previous
import functools

import jax
import jax.numpy as jnp
from jax import lax
from jax.experimental import pallas as pl
from jax.experimental.pallas import tpu as pltpu
from jax.experimental.pallas import tpu_sc as plsc

N = 50000
E = 800000
B = 64
F_IN = 128
H = 64
HEADS = 4
DH = 16
NC = 8

NCORE = 2
NSUB = 16
HALF = N // NCORE
EPW = E // NSUB

WACC = 72
ACC_ROWS = 25088
SLAB = ACC_ROWS // NSUB
DUMMY = 25080
CH_E = 80
ST = 400
SUB = ST // CH_E
CH_P = 2000
CAP = 51200

BLK = 1000
GRID = N // BLK


def _partition_body(src_hbm, dst_hbm, edgep, counts, sbuf, dbuf, eacc, cbuf):
    c = lax.axis_index("c")
    s = lax.axis_index("s")
    w = c * NSUB + s
    lo = c * HALF

    cnt = jnp.int32(0)
    for ch in range(E // NSUB // CH_P):
        g = ch * NSUB + s
        pltpu.sync_copy(src_hbm.at[pl.ds(g * CH_P, CH_P)], sbuf)
        pltpu.sync_copy(dst_hbm.at[pl.ds(g * CH_P, CH_P)], dbuf)

        @pl.loop(0, CH_P // 16, init_carry=cnt)
        def _vloop(k, cnt):
            sv = sbuf[pl.ds(k * 16, 16)]
            dv = dbuf[pl.ds(k * 16, 16)]
            mask = (dv >= lo) & (dv < lo + HALF)
            key = jnp.where(mask, 0, 1)
            val = jnp.bitwise_or(lax.shift_left(sv, 16), dv)
            _, vs = plsc.sort_key_val(key, val)
            eacc[pl.ds(cnt, 16)] = vs
            return cnt + jnp.sum(mask.astype(jnp.int32))

        cnt = _vloop

    dummyv = (lo + DUMMY) + jnp.zeros((16,), jnp.int32)
    for q in range(ST // 16):
        eacc[pl.ds(cnt + q * 16, 16)] = dummyv
    cntpad = ((cnt + (ST - 1)) // ST) * ST

    @pl.loop(0, (cntpad + 2047) // 2048)
    def _flush(j):
        pltpu.sync_copy(eacc.at[pl.ds(j * 2048, 2048)],
                        edgep.at[pl.ds(w * CAP + j * 2048, 2048)])

    cbuf[...] = cntpad + jnp.zeros((16,), jnp.int32)
    pltpu.sync_copy(cbuf, counts.at[pl.ds(w * 16, 16)])


@functools.cache
def _get_partition():
    mesh = plsc.VectorSubcoreMesh(core_axis_name="c", subcore_axis_name="s",
                                  num_cores=NCORE, num_subcores=NSUB)
    return pl.kernel(
        _partition_body,
        out_type=(
            jax.ShapeDtypeStruct((NCORE * NSUB * CAP,), jnp.int32),
            jax.ShapeDtypeStruct((NCORE * NSUB * 16,), jnp.int32),
        ),
        mesh=mesh,
        compiler_params=pltpu.CompilerParams(needs_layout_passes=False,
                                             use_tc_tiling_on_sc=False),
        scratch_types=[
            pltpu.VMEM((CH_P,), jnp.int32),
            pltpu.VMEM((CH_P,), jnp.int32),
            pltpu.VMEM((CAP,), jnp.int32),
            pltpu.VMEM((16,), jnp.int32),
        ],
    )


def _edge_body(hp_hbm, ad_hbm, edgep, counts, zeros_hbm, out_hbm,
               acc, ebuf, hp0, hp1, ad0, ad1, si0, si1, ai0, ai1, di0, di1,
               cbuf, sg0, sg1, ss0, ss1):
    c = lax.axis_index("c")
    s = lax.axis_index("s")
    w = c * NSUB + s
    lo = c * HALF

    pltpu.sync_copy(zeros_hbm, acc.at[pl.ds(s * SLAB, SLAB)])
    pltpu.sync_copy(counts.at[pl.ds(w * 16, 16)], cbuf)
    plsc.subcore_barrier()
    cnt = cbuf[...][0]

    hpb = (hp0, hp1)
    adb = (ad0, ad1)
    sib = (si0, si1)
    aib = (ai0, ai1)
    dib = (di0, di1)
    sgs = (sg0, sg1)
    sss = (ss0, ss1)
    iot = lax.iota(jnp.int32, 16)

    def unpack(k, p):
        for g in range(CH_E // 16):
            ev = ebuf[pl.ds(k * CH_E + g * 16, 16)]
            dg = jnp.bitwise_and(ev, 65535)
            sib[p][pl.ds(g * 16, 16)] = lax.shift_right_logical(ev, 16)
            aib[p][pl.ds(g * 16, 16)] = jnp.minimum(dg, N - 1)
            dib[p][pl.ds(g * 16, 16)] = dg - lo

    def fire_gathers(p):
        g = pltpu.async_copy(hp_hbm.at[sib[p]], hpb[p], sgs[p])
        a = pltpu.async_copy(ad_hbm.at[aib[p]], adb[p], sgs[p])
        return g, a

    @pl.loop(0, cnt // ST)
    def _round(r):
        pltpu.sync_copy(edgep.at[pl.ds(w * CAP + r * ST, ST)], ebuf)

        unpack(0, 0)
        gat = fire_gathers(0)
        scat = [None, None]
        for k in range(SUB):
            p = k % 2
            if k + 1 < SUB:
                if scat[1 - p] is not None:
                    scat[1 - p].wait()
                    scat[1 - p] = None
                unpack(k + 1, 1 - p)
                nxt = fire_gathers(1 - p)
            gat[0].wait()
            gat[1].wait()
            buf = hpb[p]
            for g in range(CH_E // 16):
                rows = iot + g * 16
                for j in range(HEADS):
                    asv = plsc.load_gather(
                        buf, [rows, jnp.full((16,), H + j, jnp.int32)])
                    adv = plsc.load_gather(
                        adb[p], [rows, jnp.full((16,), j, jnp.int32)])
                    e = asv + adv
                    e = jnp.where(e >= 0.0, e, 0.2 * e)
                    ex = jnp.exp(e)
                    plsc.store_scatter(
                        buf, [rows, jnp.full((16,), H + j, jnp.int32)], ex)

            @pl.loop(0, CH_E, unroll=4)
            def _scale(ei):
                tailv = buf[ei, pl.ds(WACC - 16, 16)]
                for j in range(HEADS):
                    exs = tailv[H + j - (WACC - 16)]
                    buf[ei, pl.ds(DH * j, DH)] = (
                        buf[ei, pl.ds(DH * j, DH)] * exs)

            if scat[p] is not None:
                scat[p].wait()
            scat[p] = pltpu.async_copy(buf, acc.at[dib[p]], sss[p], add=True)
            if k + 1 < SUB:
                gat = nxt
        for d in scat:
            if d is not None:
                d.wait()

    plsc.subcore_barrier()

    @pl.when(s < NSUB - 1)
    def _():
        pltpu.sync_copy(acc.at[pl.ds(s * SLAB, SLAB)],
                        out_hbm.at[pl.ds(c * HALF + s * SLAB, SLAB)])

    @pl.when(s == NSUB - 1)
    def _():
        pltpu.sync_copy(acc.at[pl.ds((NSUB - 1) * SLAB, HALF - (NSUB - 1) * SLAB)],
                        out_hbm.at[pl.ds(c * HALF + (NSUB - 1) * SLAB,
                                         HALF - (NSUB - 1) * SLAB)])


@functools.cache
def _get_edge():
    mesh = plsc.VectorSubcoreMesh(core_axis_name="c", subcore_axis_name="s",
                                  num_cores=NCORE, num_subcores=NSUB)
    return pl.kernel(
        _edge_body,
        out_type=jax.ShapeDtypeStruct((N, WACC), jnp.float32),
        mesh=mesh,
        compiler_params=pltpu.CompilerParams(needs_layout_passes=False,
                                             use_tc_tiling_on_sc=False),
        scratch_types=[
            pltpu.VMEM_SHARED((ACC_ROWS, WACC), jnp.float32),
            pltpu.VMEM((ST,), jnp.int32),
            pltpu.VMEM((CH_E, WACC), jnp.float32),
            pltpu.VMEM((CH_E, WACC), jnp.float32),
            pltpu.VMEM((CH_E, DH), jnp.float32),
            pltpu.VMEM((CH_E, DH), jnp.float32),
            pltpu.VMEM((CH_E,), jnp.int32),
            pltpu.VMEM((CH_E,), jnp.int32),
            pltpu.VMEM((CH_E,), jnp.int32),
            pltpu.VMEM((CH_E,), jnp.int32),
            pltpu.VMEM((CH_E,), jnp.int32),
            pltpu.VMEM((CH_E,), jnp.int32),
            pltpu.VMEM((16,), jnp.int32),
            pltpu.SemaphoreType.DMA,
            pltpu.SemaphoreType.DMA,
            pltpu.SemaphoreType.DMA,
            pltpu.SemaphoreType.DMA,
        ],
    )


def _heads_expand(v4):
    return jnp.concatenate(
        [jnp.broadcast_to(v4[:, j:j + 1], (v4.shape[0], DH)) for j in
         range(HEADS)], axis=1)


def _project(h, w_ref, as_ref, ad_ref, hpe_ref, atd_ref):
    hp = jnp.dot(h, w_ref[...], preferred_element_type=jnp.float32)
    a_s = jnp.dot(hp, as_ref[...], preferred_element_type=jnp.float32)
    a_d = jnp.dot(hp, ad_ref[...], preferred_element_type=jnp.float32)
    hpe_ref[...] = jnp.concatenate(
        [hp, a_s, jnp.zeros((BLK, WACC - H - HEADS), jnp.float32)], axis=1)
    atd_ref[...] = jnp.concatenate(
        [a_d, jnp.zeros((BLK, DH - HEADS), jnp.float32)], axis=1)


def _prologue_body(x_ref, ew_ref, eb_ref, w_ref, as_ref, ad_ref,
                   hpe_ref, atd_ref):
    h0 = jnp.maximum(
        jnp.dot(x_ref[...], ew_ref[...], preferred_element_type=jnp.float32)
        + eb_ref[...], 0.0)
    _project(h0, w_ref, as_ref, ad_ref, hpe_ref, atd_ref)


def _mid_body(un_ref, b_ref, w_ref, as_ref, ad_ref, hpe_ref, atd_ref):
    un = un_ref[...]
    rec = 1.0 / (un[:, H:H + HEADS] + 1e-16)
    h = jnp.maximum(un[:, :H] * _heads_expand(rec) + b_ref[...], 0.0)
    _project(h, w_ref, as_ref, ad_ref, hpe_ref, atd_ref)


def _wspec(shape):
    return pl.BlockSpec(shape, lambda i: (0,) * len(shape))


_prologue = pl.pallas_call(
    _prologue_body,
    grid=(GRID,),
    in_specs=[
        pl.BlockSpec((BLK, F_IN), lambda i: (i, 0)),
        _wspec((F_IN, H)), _wspec((1, H)), _wspec((H, H)),
        _wspec((H, HEADS)), _wspec((H, HEADS)),
    ],
    out_specs=[
        pl.BlockSpec((BLK, WACC), lambda i: (i, 0)),
        pl.BlockSpec((BLK, DH), lambda i: (i, 0)),
    ],
    out_shape=[
        jax.ShapeDtypeStruct((N, WACC), jnp.float32),
        jax.ShapeDtypeStruct((N, DH), jnp.float32),
    ],
)

_mid = pl.pallas_call(
    _mid_body,
    grid=(GRID,),
    in_specs=[
        pl.BlockSpec((BLK, WACC), lambda i: (i, 0)),
        _wspec((1, H)), _wspec((H, H)),
        _wspec((H, HEADS)), _wspec((H, HEADS)),
    ],
    out_specs=[
        pl.BlockSpec((BLK, WACC), lambda i: (i, 0)),
        pl.BlockSpec((BLK, DH), lambda i: (i, 0)),
    ],
    out_shape=[
        jax.ShapeDtypeStruct((N, WACC), jnp.float32),
        jax.ShapeDtypeStruct((N, DH), jnp.float32),
    ],
)


def _final_body(un_ref, b_ref, bf_ref, sums_ref, maxs_ref, cnts_ref):
    i = pl.program_id(0)

    @pl.when(i == 0)
    def _():
        sums_ref[...] = jnp.zeros((B, H), jnp.float32)
        maxs_ref[...] = jnp.full((B, H), -jnp.inf, jnp.float32)
        cnts_ref[...] = jnp.zeros((B, H), jnp.float32)

    un = un_ref[...]
    rec = 1.0 / (un[:, H:H + HEADS] + 1e-16)
    h = jnp.maximum(un[:, :H] * _heads_expand(rec) + b_ref[...], 0.0)
    bf = bf_ref[...]
    blo = bf_ref[0, 0]
    bhi = bf_ref[BLK - 1, 0]
    for b in range(B):
        @pl.when((blo <= float(b)) & (float(b) <= bhi))
        def _():
            m = bf == float(b)
            mb = jnp.max(jnp.where(m, h, -jnp.inf), axis=0, keepdims=True)
            maxs_ref[b:b + 1, :] = jnp.maximum(maxs_ref[b:b + 1, :], mb)
            sums_ref[b:b + 1, :] += jnp.sum(
                jnp.where(m, h, 0.0), axis=0, keepdims=True)
            cnts_ref[b:b + 1, :] += jnp.full(
                (1, H), jnp.sum(m.astype(jnp.float32)))


_final = pl.pallas_call(
    _final_body,
    grid=(GRID,),
    in_specs=[
        pl.BlockSpec((BLK, WACC), lambda i: (i, 0)),
        _wspec((1, H)),
        pl.BlockSpec((BLK, 1), lambda i: (i, 0)),
    ],
    out_specs=[_wspec((B, H)), _wspec((B, H)), _wspec((B, H))],
    out_shape=[
        jax.ShapeDtypeStruct((B, H), jnp.float32),
        jax.ShapeDtypeStruct((B, H), jnp.float32),
        jax.ShapeDtypeStruct((B, H), jnp.float32),
    ],
)


def _ln(x, g, b):
    m = jnp.mean(x, axis=-1, keepdims=True)
    v = jnp.mean((x - m) ** 2, axis=-1, keepdims=True)
    return (x - m) / jnp.sqrt(v + 1e-5) * g + b


def _head_body(sums_ref, maxs_ref, cnts_ref, ts_ref,
               tw1_ref, tb1_ref, tg_ref, tbb_ref, tw2_ref, tb2_ref,
               cg_ref, cb_ref, cw1_ref, cb1_ref, cw2_ref, cb2_ref,
               out_ref):
    mean = sums_ref[...] / jnp.clip(cnts_ref[...], 1.0, None)
    t = jnp.dot(ts_ref[...], tw1_ref[...],
                preferred_element_type=jnp.float32) + tb1_ref[...]
    t = _ln(t, tg_ref[...], tbb_ref[...])
    t = jnp.maximum(t, 0.0)
    t = jnp.dot(t, tw2_ref[...],
                preferred_element_type=jnp.float32) + tb2_ref[...]
    pooled = jnp.concatenate([mean, maxs_ref[...], t], axis=1)
    z = _ln(pooled, cg_ref[...], cb_ref[...])
    z = jnp.maximum(
        jnp.dot(z, cw1_ref[...], preferred_element_type=jnp.float32)
        + cb1_ref[...], 0.0)
    out_ref[...] = jnp.dot(
        z, cw2_ref[...], preferred_element_type=jnp.float32) + cb2_ref[...]


_POOL = 2 * H + H // 2

_head = pl.pallas_call(
    _head_body,
    out_shape=jax.ShapeDtypeStruct((B, NC), jnp.float32),
)


def _att_mat(att):
    return (att[:, :, None] * jnp.eye(HEADS, dtype=jnp.float32)[:, None, :]
            ).reshape(H, HEADS)


def kernel(x, edge_index, batch, trackster_features, params):
    p = params
    src, dst = edge_index[0], edge_index[1]
    zrows = jnp.zeros((SLAB, WACC), jnp.float32)
    edgep, counts = _get_partition()(src, dst)
    edge_call = _get_edge()

    convs = p['convs']
    as_m = [_att_mat(c['att_src']) for c in convs]
    ad_m = [_att_mat(c['att_dst']) for c in convs]

    hpe, atd = _prologue(x, p['enc_W'], p['enc_b'].reshape(1, H),
                         convs[0]['W'], as_m[0], ad_m[0])
    un = None
    for L in range(3):
        un = edge_call(hpe, atd, edgep, counts, zrows)
        if L < 2:
            hpe, atd = _mid(un, convs[L]['b'].reshape(1, H),
                            convs[L + 1]['W'], as_m[L + 1], ad_m[L + 1])

    sums, maxs, cnts = _final(un, convs[2]['b'].reshape(1, H),
                              batch.astype(jnp.float32).reshape(N, 1))

    return _head(
        sums, maxs, cnts, trackster_features,
        p['ts_W1'], p['ts_b1'].reshape(1, H // 2),
        p['ts_ln_g'].reshape(1, H // 2), p['ts_ln_b'].reshape(1, H // 2),
        p['ts_W2'], p['ts_b2'].reshape(1, H // 2),
        p['cl_ln_g'].reshape(1, _POOL), p['cl_ln_b'].reshape(1, _POOL),
        p['cl_W1'], p['cl_b1'].reshape(1, H),
        p['cl_W2'], p['cl_b2'].reshape(1, NC),
    )
